# baseline (speedup 1.0000x reference)
import functools

import jax
import jax.numpy as jnp
from jax import lax
from jax.experimental import pallas as pl
from jax.experimental.pallas import tpu as pltpu
from jax.experimental.pallas import tpu_sc as plsc

B, NBFEAT, H, W = 8, 384, 128, 128
ENC, P = 64, 1024
_EPSILON = 1e-06
HS = 16
NH = H // HS
HW = H * W


def _conv_body(fm_ref, w1t_ref, wp_ref, wb_ref,
               pfmt_ref, x_ref, bf_ref, bl_ref, acc_ref):
    h = pl.program_id(1)
    fm2 = fm_ref[0]
    pfmt = lax.dot_general(fm2, w1t_ref[...],
                           (((0,), (0,)), ((), ())),
                           preferred_element_type=jnp.float32)
    pfmt_ref[0] = pfmt
    x = jnp.dot(pfmt, wp_ref[...], preferred_element_type=jnp.float32)
    x_ref[0] = x
    part = fm2[:, 0:W]
    for j in range(1, HS * W // W):
        part = part + fm2[:, j * W:(j + 1) * W]
    @pl.when(h == 0)
    def _():
        acc_ref[...] = part
    @pl.when(h > 0)
    def _():
        acc_ref[...] += part

    @pl.when(h == NH - 1)
    def _():
        bfv = jnp.sum(acc_ref[...], axis=1, keepdims=True) / jnp.float32(HW)
        bf_ref[0] = bfv
        bl = jnp.maximum(jnp.sum(bfv * wb_ref[0]), 0.0)
        bl_ref[...] = jnp.full((1, 1, 128), bl, dtype=jnp.float32)


def _conv_stage(featureMaps, W1, Wp, Wb):
    grid = (B, NH)
    out = pl.pallas_call(
        _conv_body,
        grid=grid,
        in_specs=[
            pl.BlockSpec((1, NBFEAT, HS * W), lambda b, h: (b, 0, h)),
            pl.BlockSpec((NBFEAT, ENC), lambda b, h: (0, 0)),
            pl.BlockSpec((ENC, 1), lambda b, h: (0, 0)),
            pl.BlockSpec((1, NBFEAT, 1), lambda b, h: (0, 0, 0)),
        ],
        out_specs=[
            pl.BlockSpec((1, HS * W, ENC), lambda b, h: (b, h, 0)),
            pl.BlockSpec((1, HS * W, 1), lambda b, h: (b, h, 0)),
            pl.BlockSpec((1, NBFEAT, 1), lambda b, h: (b, 0, 0)),
            pl.BlockSpec((1, 1, 128), lambda b, h: (b, 0, 0)),
        ],
        out_shape=[
            jax.ShapeDtypeStruct((B, HW, ENC), jnp.float32),
            jax.ShapeDtypeStruct((B, HW, 1), jnp.float32),
            jax.ShapeDtypeStruct((B, NBFEAT, 1), jnp.float32),
            jax.ShapeDtypeStruct((B, 1, 128), jnp.float32),
        ],
        scratch_shapes=[pltpu.VMEM((NBFEAT, W), jnp.float32)],
    )(featureMaps.reshape(B, NBFEAT, HW), W1.T, Wp.T, Wb.reshape(1, NBFEAT, 1))
    return out


_SC_INFO = None


def _sc_gather(table, idx):
    info = plsc.get_sparse_core_info()
    nw = info.num_cores * info.num_subcores
    n = idx.shape[0]
    b_per_w = n // nw
    mesh = plsc.VectorSubcoreMesh(core_axis_name="c", subcore_axis_name="s")

    @functools.partial(
        pl.kernel, mesh=mesh,
        out_type=jax.ShapeDtypeStruct((n, 2 * ENC), jnp.float32),
        scratch_types=[
            pltpu.VMEM((b_per_w,), jnp.int32),
            pltpu.VMEM((b_per_w, 2 * ENC), jnp.float32),
            pltpu.SemaphoreType.DMA,
        ],
    )
    def k(table_hbm, idx_hbm, out_hbm, idx_v, rows_v, sem):
        wid = lax.axis_index("s") * info.num_cores + lax.axis_index("c")
        base = wid * b_per_w
        pltpu.sync_copy(idx_hbm.at[pl.ds(base, b_per_w)], idx_v)
        pltpu.async_copy(table_hbm.at[idx_v], rows_v, sem).wait()
        pltpu.sync_copy(rows_v, out_hbm.at[pl.ds(base, b_per_w)])

    return k(table, idx)


def kernel(featureMaps, W1, b1, Wp, bp, Wb, bb):
    pfmt, x_flat, baseFeat3, bl_pad = _conv_stage(featureMaps, W1, Wp, Wb)
    x_full = x_flat.reshape(B, H, W)
    baseFeat = baseFeat3.reshape(B, NBFEAT)
    baseline = bl_pad.reshape(B, 128)[:, :1]
    x = x_full[:, 3:-3, 3:-3] + (jnp.dot(Wp, b1) + bp)[0]
    h, w = H - 6, W - 6
    flatX = jax.nn.sigmoid(x.reshape(B, h * w))
    probs = flatX / (flatX.sum(axis=1, keepdims=True) + _EPSILON)
    _, flatInds = jax.lax.top_k(probs, P)
    abs_i = flatInds % w
    ord_i = flatInds // w
    full_idx = (ord_i + 3) * W + (abs_i + 3) + (jnp.arange(B, dtype=jnp.int32) * HW)[:, None]
    table2 = pfmt.reshape(B * HW // 2, 2 * ENC)
    fi = full_idx.reshape(B * P)
    pairs = _sc_gather(table2, fi // 2)
    parity = (fi % 2)[:, None]
    pf_rows = jnp.where(parity == 1, pairs[:, ENC:], pairs[:, :ENC]) + b1[None, :]
    pointFeat = pf_rows.reshape(B, P, ENC)
    depth = jnp.zeros((B, P, 1), dtype=jnp.float32)
    absf = abs_i[..., None].astype(jnp.float32)
    ordf = ord_i[..., None].astype(jnp.float32)
    points_full = jnp.concatenate([absf, ordf, depth, pointFeat], axis=-1)
    batch = jnp.repeat(jnp.arange(B), P)
    pos = jnp.concatenate([absf, ordf, depth], axis=-1).reshape(B * P, 3)
    pointfeatures = pf_rows
    return (points_full, batch, pos, pointfeatures, probs, flatInds,
            baseFeat, baseline)
